# baseline (speedup 1.0000x reference)
import functools

import jax
import jax.numpy as jnp
from jax import lax
from jax.experimental import pallas as pl
from jax.experimental.pallas import tpu as pltpu
from jax.experimental.pallas import tpu_sc as plsc

HIDDEN = 64
CHUNK = 128
TC_BLK = 8192


def _make_gather(B, n_workers, b_per_w, n_chunks):
    mesh = plsc.VectorSubcoreMesh(core_axis_name="c", subcore_axis_name="s")
    n_pairs = n_chunks // 2

    @functools.partial(
        pl.kernel,
        mesh=mesh,
        out_type=jax.ShapeDtypeStruct((B, HIDDEN), jnp.float32),
        scratch_types=[
            pltpu.VMEM((n_chunks, CHUNK), jnp.int32),
            pltpu.VMEM((2, CHUNK, HIDDEN), jnp.float32),
            pltpu.SemaphoreType.DMA,
            pltpu.SemaphoreType.DMA,
        ],
        compiler_params=pltpu.CompilerParams(use_tc_tiling_on_sc=False),
    )
    def gather_k(idx_hbm, table_hbm, out_hbm, idx_v, rows_v, sem0, sem1):
        nc = lax.axis_size("c")
        wid = lax.axis_index("s") * nc + lax.axis_index("c")
        base = wid * b_per_w

        pltpu.sync_copy(idx_hbm.at[wid], idx_v)

        def start(j, buf, sem):
            return pltpu.async_copy(table_hbm.at[idx_v.at[j]],
                                    rows_v.at[buf], sem)

        def wait(j, buf, sem):
            pltpu.make_async_copy(table_hbm.at[idx_v.at[j]],
                                  rows_v.at[buf], sem).wait()

        def store(j, buf):
            pltpu.sync_copy(rows_v.at[buf],
                            out_hbm.at[pl.ds(base + j * CHUNK, CHUNK)])

        start(0, 0, sem0)

        def body(g, carry):
            j0 = 2 * g
            start(j0 + 1, 1, sem1)
            wait(j0, 0, sem0)
            store(j0, 0)
            start(j0 + 2, 0, sem0)
            wait(j0 + 1, 1, sem1)
            store(j0 + 1, 1)
            return carry

        lax.fori_loop(0, n_pairs - 1, body, 0)

        j0 = n_chunks - 2
        start(j0 + 1, 1, sem1)
        wait(j0, 0, sem0)
        store(j0, 0)
        wait(j0 + 1, 1, sem1)
        store(j0 + 1, 1)

    return gather_k


def _tc_body(x_ref, w_ref, b_ref, o_ref):
    x = jnp.maximum(x_ref[...], 0.0)
    o_ref[...] = lax.dot_general(
        x, w_ref[...], (((1,), (1,)), ((), ())),
        preferred_element_type=jnp.float32) + b_ref[...]


def kernel(inputs, table, W, b):
    batch, seq = inputs.shape
    B = batch * seq
    info = plsc.get_sparse_core_info()
    n_workers = info.num_cores * info.num_subcores
    b_per_w = B // n_workers
    n_chunks = b_per_w // CHUNK

    idx = inputs.reshape(n_workers, n_chunks, CHUNK).astype(jnp.int32)
    gathered = _make_gather(B, n_workers, b_per_w, n_chunks)(idx, table)

    out = pl.pallas_call(
        _tc_body,
        grid=(B // TC_BLK,),
        in_specs=[
            pl.BlockSpec((TC_BLK, HIDDEN), lambda i: (i, 0)),
            pl.BlockSpec((HIDDEN, HIDDEN), lambda i: (0, 0)),
            pl.BlockSpec((1, HIDDEN), lambda i: (0, 0)),
        ],
        out_specs=pl.BlockSpec((TC_BLK, HIDDEN), lambda i: (i, 0)),
        out_shape=jax.ShapeDtypeStruct((B, HIDDEN), jnp.float32),
    )(gathered, W, b.reshape(1, HIDDEN))

    return out.reshape(batch, seq, HIDDEN)


# speedup vs baseline: 1.3057x; 1.3057x over previous
import functools

import jax
import jax.numpy as jnp
from jax import lax
from jax.experimental import pallas as pl
from jax.experimental.pallas import tpu as pltpu
from jax.experimental.pallas import tpu_sc as plsc

HIDDEN = 64
PAIR = 64
TCF_BLK = 4096
N_TABLE = 1000000


def _make_gather(B, n_workers, b_per_w, n_chunks):
    mesh = plsc.VectorSubcoreMesh(core_axis_name="c", subcore_axis_name="s")
    n_pairs = n_chunks // 2
    pairs_per_w = b_per_w // 2

    @functools.partial(
        pl.kernel,
        mesh=mesh,
        out_type=jax.ShapeDtypeStruct((B // 2, 2 * HIDDEN), jnp.float32),
        scratch_types=[
            pltpu.VMEM((n_chunks, 2 * PAIR), jnp.int32),
            pltpu.VMEM((2, 2, PAIR, HIDDEN), jnp.float32),
            pltpu.SemaphoreType.DMA,
            pltpu.SemaphoreType.DMA,
        ],
        compiler_params=pltpu.CompilerParams(use_tc_tiling_on_sc=False),
    )
    def gather_k(idx_hbm, table_hbm, out_hbm, idx_v, rows_v, sem0, sem1):
        nc = lax.axis_size("c")
        wid = lax.axis_index("s") * nc + lax.axis_index("c")
        pair_base = wid * pairs_per_w

        pltpu.sync_copy(idx_hbm.at[wid], idx_v)

        def copies(c, buf, sem):
            for g in range(2):
                src = table_hbm.at[idx_v.at[c, pl.ds(PAIR * g, PAIR)]]
                dst = rows_v.at[buf, g]
                yield src, dst, sem

        def start(c, buf, sem):
            for src, dst, s in copies(c, buf, sem):
                pltpu.async_copy(src, dst, s)

        def wait(c, buf, sem):
            for src, dst, s in copies(c, buf, sem):
                pltpu.make_async_copy(src, dst, s).wait()

        def store(c, buf):
            for g in range(2):
                pltpu.sync_copy(
                    rows_v.at[buf, g],
                    out_hbm.at[pl.ds(pair_base + c * PAIR, PAIR),
                               pl.ds(HIDDEN * g, HIDDEN)])

        start(0, 0, sem0)

        def body(i, carry):
            c0 = 2 * i
            start(c0 + 1, 1, sem1)
            wait(c0, 0, sem0)
            store(c0, 0)
            start(c0 + 2, 0, sem0)
            wait(c0 + 1, 1, sem1)
            store(c0 + 1, 1)
            return carry

        lax.fori_loop(0, n_pairs - 1, body, 0)

        c0 = n_chunks - 2
        start(c0 + 1, 1, sem1)
        wait(c0, 0, sem0)
        store(c0, 0)
        wait(c0 + 1, 1, sem1)
        store(c0 + 1, 1)

    return gather_k


def _tcf_body(x_ref, wd_ref, b_ref, o_ref):
    x = jnp.maximum(x_ref[...], 0.0)
    y = lax.dot_general(
        x, wd_ref[...], (((1,), (0,)), ((), ())),
        preferred_element_type=jnp.float32) + b_ref[...]
    o_ref[0] = y[:, :HIDDEN]
    o_ref[1] = y[:, HIDDEN:]


def kernel(inputs, table, W, b):
    batch, seq = inputs.shape
    B = batch * seq
    info = plsc.get_sparse_core_info()
    n_workers = info.num_cores * info.num_subcores
    b_per_w = B // n_workers
    n_chunks = b_per_w // (2 * PAIR)

    flat = inputs.reshape(B)
    ia = flat[:B // 2].reshape(n_workers, n_chunks, PAIR)
    ib = flat[B // 2:].reshape(n_workers, n_chunks, PAIR)
    idx2 = jnp.concatenate([ia, ib], axis=-1)

    g128 = _make_gather(B, n_workers, b_per_w, n_chunks)(idx2, table)

    wt = W.T
    z = jnp.zeros_like(wt)
    wd = jnp.block([[wt, z], [z, wt]])
    bcat = jnp.concatenate([b, b]).reshape(1, 2 * HIDDEN)

    out = pl.pallas_call(
        _tcf_body,
        grid=(B // (2 * TCF_BLK),),
        in_specs=[
            pl.BlockSpec((TCF_BLK, 2 * HIDDEN), lambda i: (i, 0)),
            pl.BlockSpec((2 * HIDDEN, 2 * HIDDEN), lambda i: (0, 0)),
            pl.BlockSpec((1, 2 * HIDDEN), lambda i: (0, 0)),
        ],
        out_specs=pl.BlockSpec((2, TCF_BLK, HIDDEN), lambda i: (0, i, 0)),
        out_shape=jax.ShapeDtypeStruct((2, B // 2, HIDDEN), jnp.float32),
    )(g128, wd, bcat)

    return out.reshape(batch, seq, HIDDEN)
